# Initial kernel scaffold; baseline (speedup 1.0000x reference)
import jax
import jax.numpy as jnp
from jax.experimental import pallas as pl


def kernel(X, edge_index, W, b):
    raise NotImplementedError("write your pallas kernel here")


# speedup vs baseline: 6.6265x; 6.6265x over previous
import functools

import jax
import jax.numpy as jnp
from jax import lax
from jax.experimental import pallas as pl
from jax.experimental.pallas import tpu as pltpu
from jax.experimental.pallas import tpu_sc as plsc

NC = 2
NS = 16
NW = NC * NS
CH = 128


def _sc_aggregate(x, src, dst, zeros):
    n, d = x.shape
    e = src.shape[0]
    ew = e // NW
    nfull = ew // CH
    rem = ew - nfull * CH
    nrc = n // CH
    nt = n - nrc * CH
    nkr = (nrc + NS - 1) // NS

    mesh = plsc.VectorSubcoreMesh(core_axis_name="c", subcore_axis_name="s")

    scratch = [
        pltpu.VMEM((CH,), jnp.int32),
        pltpu.VMEM((CH,), jnp.int32),
        pltpu.VMEM((CH, d), jnp.float32),
        pltpu.VMEM_SHARED((n, d), jnp.float32),
        pltpu.SemaphoreType.DMA,
    ]
    if rem:
        scratch += [
            pltpu.VMEM((rem,), jnp.int32),
            pltpu.VMEM((rem,), jnp.int32),
            pltpu.VMEM((rem, d), jnp.float32),
        ]
    if nt:
        scratch += [pltpu.VMEM((nt, d), jnp.float32)]

    @functools.partial(
        pl.kernel,
        out_type=jax.ShapeDtypeStruct((NC, n, d), jnp.float32),
        mesh=mesh,
        scratch_types=scratch,
    )
    def agg_kernel(x_hbm, src_hbm, dst_hbm, zeros_hbm, out_hbm, sidx, didx,
                   rows, agg_sh, sem, *tail):
        cid = lax.axis_index("c")
        sid = lax.axis_index("s")
        wid = sid * NC + cid

        for k in range(nkr):
            c = sid + NS * k

            @pl.when(c < nrc)
            def _():
                r0 = pl.multiple_of(c * CH, 8)
                pltpu.sync_copy(zeros_hbm.at[pl.ds(r0, CH)], rows)
                pltpu.sync_copy(rows, agg_sh.at[pl.ds(r0, CH)])

        if nt:
            trows = tail[-1]

            @pl.when(sid == 0)
            def _():
                pltpu.sync_copy(zeros_hbm.at[pl.ds(nrc * CH, nt)], trows)
                pltpu.sync_copy(trows, agg_sh.at[pl.ds(nrc * CH, nt)])

        plsc.subcore_barrier()

        ebase = wid * ew

        def body(j, carry):
            off = pl.multiple_of(ebase + j * CH, 8)
            pltpu.sync_copy(src_hbm.at[pl.ds(off, CH)], sidx)
            pltpu.sync_copy(dst_hbm.at[pl.ds(off, CH)], didx)
            pltpu.async_copy(x_hbm.at[sidx], rows, sem).wait()
            pltpu.sync_copy(rows, agg_sh.at[didx], add=True)
            return carry

        lax.fori_loop(0, nfull, body, 0)

        if rem:
            sidxr, didxr, rowsr = tail[0], tail[1], tail[2]
            offr = pl.multiple_of(ebase + nfull * CH, 8)
            pltpu.sync_copy(src_hbm.at[pl.ds(offr, rem)], sidxr)
            pltpu.sync_copy(dst_hbm.at[pl.ds(offr, rem)], didxr)
            pltpu.async_copy(x_hbm.at[sidxr], rowsr, sem).wait()
            pltpu.sync_copy(rowsr, agg_sh.at[didxr], add=True)

        plsc.subcore_barrier()

        for k in range(nkr):
            c = sid + NS * k

            @pl.when(c < nrc)
            def _():
                r0 = pl.multiple_of(c * CH, 8)
                pltpu.sync_copy(agg_sh.at[pl.ds(r0, CH)], rows)
                pltpu.sync_copy(rows, out_hbm.at[cid, pl.ds(r0, CH)])

        if nt:
            trows = tail[-1]

            @pl.when(sid == 0)
            def _():
                pltpu.sync_copy(agg_sh.at[pl.ds(nrc * CH, nt)], trows)
                pltpu.sync_copy(trows, out_hbm.at[cid, pl.ds(nrc * CH, nt)])

    return agg_kernel(x, src, dst, zeros)


def _tc_mlp(x, partials, w, b2d):
    n, d = x.shape
    br = 1000

    def body(x_ref, p_ref, w_ref, b_ref, o_ref):
        h = x_ref[...] + p_ref[0] + p_ref[1]
        o_ref[...] = (
            jnp.dot(h, w_ref[...], preferred_element_type=jnp.float32)
            + b_ref[...]
        )

    return pl.pallas_call(
        body,
        grid=(n // br,),
        in_specs=[
            pl.BlockSpec((br, d), lambda i: (i, 0)),
            pl.BlockSpec((NC, br, d), lambda i: (0, i, 0)),
            pl.BlockSpec((d, d), lambda i: (0, 0)),
            pl.BlockSpec((1, d), lambda i: (0, 0)),
        ],
        out_specs=pl.BlockSpec((br, d), lambda i: (i, 0)),
        out_shape=jax.ShapeDtypeStruct((n, d), jnp.float32),
    )(x, partials, w, b2d)


def kernel(X, edge_index, W, b):
    n, d = X.shape
    zeros = jnp.zeros((n, d), dtype=jnp.float32)
    partials = _sc_aggregate(X, edge_index[0], edge_index[1], zeros)
    return _tc_mlp(X, partials, W, b.reshape(1, d))


# speedup vs baseline: 12.9669x; 1.9568x over previous
import functools

import jax
import jax.numpy as jnp
from jax import lax
from jax.experimental import pallas as pl
from jax.experimental.pallas import tpu as pltpu
from jax.experimental.pallas import tpu_sc as plsc

NC = 2
NS = 16
NW = NC * NS
CH = 128


def _sc_aggregate(x, idx_all, n_pad):
    n, d = x.shape
    nwin = idx_all.shape[0]
    wpw = nwin // NW
    na = n + n_pad
    nca = na // CH
    nrc = n // CH
    nt = n - nrc * CH

    mesh = plsc.VectorSubcoreMesh(core_axis_name="c", subcore_axis_name="s")

    scratch = [
        pltpu.VMEM((2, CH), jnp.int32),
        pltpu.VMEM((2, CH), jnp.int32),
        pltpu.VMEM((2, CH), jnp.int32),
        pltpu.VMEM((2, CH), jnp.int32),
        pltpu.VMEM((CH, d), jnp.float32),
        pltpu.VMEM((CH, d), jnp.float32),
        pltpu.VMEM_SHARED((na, d), jnp.float32),
        pltpu.SemaphoreType.DMA,
        pltpu.SemaphoreType.DMA,
        pltpu.SemaphoreType.DMA,
        pltpu.SemaphoreType.DMA,
        pltpu.SemaphoreType.DMA,
        pltpu.SemaphoreType.DMA,
    ]

    @functools.partial(
        pl.kernel,
        out_type=jax.ShapeDtypeStruct((NC, n, d), jnp.float32),
        mesh=mesh,
        scratch_types=scratch,
    )
    def agg_kernel(x_hbm, idx_hbm, out_hbm, ib0, ib1, ib2, ib3, rows0,
                   rows1, agg_sh, is0, is1, is2, is3, gs0, gs1):
        cid = lax.axis_index("c")
        sid = lax.axis_index("s")
        wid = sid * NC + cid
        wbase = wid * wpw
        ibufs = (ib0, ib1, ib2, ib3)
        isems = (is0, is1, is2, is3)
        rbufs = (rows0, rows1)
        gsems = (gs0, gs1)

        def idx_req(j, t):
            pltpu.async_copy(idx_hbm.at[wbase + j], ibufs[t], isems[t])

        def idx_wait(t):
            pltpu.make_async_copy(idx_hbm.at[0], ibufs[t], isems[t]).wait()

        def gather(t_idx, t_row):
            pltpu.async_copy(x_hbm.at[ibufs[t_idx].at[0]], rbufs[t_row],
                             gsems[t_row])

        def rows_wait(t_row):
            pltpu.make_async_copy(x_hbm.at[pl.ds(0, CH)], rbufs[t_row],
                                  gsems[t_row]).wait()

        def scat(t_idx, t_row):
            pltpu.sync_copy(rbufs[t_row], agg_sh.at[ibufs[t_idx].at[1]],
                            add=True)

        for j in range(3):
            idx_req(j, j)

        zv = jnp.zeros((16,), jnp.float32)

        def zrow(i, carry):
            for t in range(d // 16):
                rows0[i, pl.ds(t * 16, 16)] = zv
            return carry

        lax.fori_loop(0, CH, zrow, 0)
        for k in range(nca // NS):
            c = sid * (nca // NS) + k
            r0 = pl.multiple_of(c * CH, 8)
            pltpu.sync_copy(rows0, agg_sh.at[pl.ds(r0, CH)])
        plsc.subcore_barrier()

        idx_wait(0)
        gather(0, 0)

        def body(k, carry):
            for t in range(4):
                j = 4 * k + t

                @pl.when(j + 1 < wpw)
                def _():
                    idx_wait((t + 1) % 4)
                    gather((t + 1) % 4, (t + 1) % 2)

                rows_wait(t % 2)
                scat(t, t % 2)

                @pl.when(j + 3 < wpw)
                def _():
                    idx_req(j + 3, (t + 3) % 4)
            return carry

        lax.fori_loop(0, wpw // 4, body, 0)
        plsc.subcore_barrier()

        for k in range((nrc + NS - 1) // NS):
            c = sid + NS * k

            @pl.when(c < nrc)
            def _():
                r0 = pl.multiple_of(c * CH, 8)
                pltpu.sync_copy(agg_sh.at[pl.ds(r0, CH)], rows0)
                pltpu.sync_copy(rows0, out_hbm.at[cid, pl.ds(r0, CH)])

        if nt:

            @pl.when(sid == 0)
            def _():
                r0 = nrc * CH
                pltpu.sync_copy(agg_sh.at[pl.ds(r0, nt)],
                                rows1.at[pl.ds(0, nt)])
                pltpu.sync_copy(rows1.at[pl.ds(0, nt)],
                                out_hbm.at[cid, pl.ds(r0, nt)])

    return agg_kernel(x, idx_all)


def _tc_mlp(x, partials, w, b2d):
    n, d = x.shape
    br = 1000

    def body(x_ref, p_ref, w_ref, b_ref, o_ref):
        h = x_ref[...] + p_ref[0] + p_ref[1]
        o_ref[...] = (
            jnp.dot(h, w_ref[...], preferred_element_type=jnp.float32)
            + b_ref[...]
        )

    return pl.pallas_call(
        body,
        grid=(n // br,),
        in_specs=[
            pl.BlockSpec((br, d), lambda i: (i, 0)),
            pl.BlockSpec((NC, br, d), lambda i: (0, i, 0)),
            pl.BlockSpec((d, d), lambda i: (0, 0)),
            pl.BlockSpec((1, d), lambda i: (0, 0)),
        ],
        out_specs=pl.BlockSpec((br, d), lambda i: (i, 0)),
        out_shape=jax.ShapeDtypeStruct((n, d), jnp.float32),
    )(x, partials, w, b2d)


def kernel(X, edge_index, W, b):
    n, d = X.shape
    e = edge_index.shape[1]
    n_pad_rows = -(-(n + 240) // CH) * CH - n
    wpw = -(-e // (NW * CH * 4)) * 4
    e_pad = wpw * NW * CH - e
    pad_ar = jnp.arange(e_pad, dtype=jnp.int32)
    src = jnp.concatenate([edge_index[0], pad_ar % n]).reshape(-1, CH)
    dst = jnp.concatenate(
        [edge_index[1], n + pad_ar % n_pad_rows]).reshape(-1, CH)
    idx_all = jnp.stack([src, dst], axis=1)
    partials = _sc_aggregate(X, idx_all, n_pad_rows)
    return _tc_mlp(X, partials, W, b.reshape(1, d))
